# baseline (speedup 1.0000x reference)
import functools

import jax
import jax.numpy as jnp
from jax import lax
from jax.experimental import pallas as pl
from jax.experimental.pallas import tpu as pltpu
from jax.experimental.pallas import tpu_sc as plsc

_NC = 2
_NS = 16
_NW = _NC * _NS

_CHUNK = 128
_NBUF = 4


def _embed_gather(total_rows, features, table_hbm, idx_hbm, out_hbm,
                  idx_v, rows_v, gsems, ssems):
    chunks_total = total_rows // _CHUNK
    cpw = chunks_total // _NW
    wid = lax.axis_index("s") * _NC + lax.axis_index("c")
    chunk0 = wid * cpw

    pltpu.sync_copy(idx_hbm.at[pl.ds(chunk0, cpw)], idx_v)

    def gather_start(j, slot):
        pltpu.async_copy(table_hbm.at[idx_v.at[j]], rows_v.at[slot],
                         gsems.at[slot])

    def gather_wait(j, slot):
        pltpu.make_async_copy(table_hbm.at[idx_v.at[j]], rows_v.at[slot],
                              gsems.at[slot]).wait()

    def store_start(j, slot):
        base = (chunk0 + j) * _CHUNK
        pltpu.async_copy(rows_v.at[slot], out_hbm.at[pl.ds(base, _CHUNK)],
                         ssems.at[slot])

    def store_wait(j, slot):
        base = (chunk0 + j) * _CHUNK
        pltpu.make_async_copy(rows_v.at[slot],
                              out_hbm.at[pl.ds(base, _CHUNK)],
                              ssems.at[slot]).wait()

    for b in range(_NBUF):
        gather_start(b, b)

    @pl.loop(0, cpw, step=_NBUF)
    def _round(j0):
        for b in range(_NBUF):
            gather_wait(j0 + b, b)
            store_start(j0 + b, b)
        for b in range(_NBUF):
            nxt = j0 + b + _NBUF

            @pl.when(nxt < cpw)
            def _():
                store_wait(j0 + b, b)
                gather_start(nxt, b)

    last0 = cpw - _NBUF
    for b in range(_NBUF):
        store_wait(last0 + b, b)


def kernel(inputs, num_embeddings, features, embedding):
    batch, hist = inputs.shape
    feat = embedding.shape[1]
    total = batch * hist
    idx2d = inputs.reshape(total // _CHUNK, _CHUNK)

    cpw = (total // _CHUNK) // _NW
    mesh = plsc.VectorSubcoreMesh(core_axis_name="c", subcore_axis_name="s",
                                  num_cores=_NC, num_subcores=_NS)
    out = pl.kernel(
        functools.partial(_embed_gather, total, feat),
        out_type=jax.ShapeDtypeStruct((total, feat), jnp.float32),
        mesh=mesh,
        scratch_types=[
            pltpu.VMEM((cpw, _CHUNK), jnp.int32),
            pltpu.VMEM((_NBUF, _CHUNK, feat), jnp.float32),
            pltpu.SemaphoreType.DMA((_NBUF,)),
            pltpu.SemaphoreType.DMA((_NBUF,)),
        ],
        compiler_params=pltpu.CompilerParams(use_tc_tiling_on_sc=False),
    )(embedding, idx2d)
    return out.reshape(batch, hist, feat)


# speedup vs baseline: 1.0019x; 1.0019x over previous
import functools

import jax
import jax.numpy as jnp
from jax import lax
from jax.experimental import pallas as pl
from jax.experimental.pallas import tpu as pltpu
from jax.experimental.pallas import tpu_sc as plsc

_NC = 2
_NS = 16
_NW = _NC * _NS

_CHUNK = 128
_NBUF = 8


def _embed_gather(total_rows, features, table_hbm, idx_hbm, out_hbm,
                  idx_v, rows_v, gsems, ssems):
    chunks_total = total_rows // _CHUNK
    cpw = chunks_total // _NW
    wid = lax.axis_index("s") * _NC + lax.axis_index("c")
    chunk0 = wid * cpw

    pltpu.sync_copy(idx_hbm.at[pl.ds(chunk0, cpw)], idx_v)

    def gather_start(j, slot):
        pltpu.async_copy(table_hbm.at[idx_v.at[j]], rows_v.at[slot],
                         gsems.at[slot])

    def gather_wait(j, slot):
        pltpu.make_async_copy(table_hbm.at[idx_v.at[j]], rows_v.at[slot],
                              gsems.at[slot]).wait()

    def store_start(j, slot):
        base = (chunk0 + j) * _CHUNK
        pltpu.async_copy(rows_v.at[slot], out_hbm.at[pl.ds(base, _CHUNK)],
                         ssems.at[slot])

    def store_wait(j, slot):
        base = (chunk0 + j) * _CHUNK
        pltpu.make_async_copy(rows_v.at[slot],
                              out_hbm.at[pl.ds(base, _CHUNK)],
                              ssems.at[slot]).wait()

    for b in range(_NBUF):
        gather_start(b, b)

    @pl.loop(0, cpw - _NBUF, step=_NBUF)
    def _round(j0):
        for b in range(_NBUF):
            gather_wait(j0 + b, b)
            store_start(j0 + b, b)
        for b in range(_NBUF):
            store_wait(j0 + b, b)
            gather_start(j0 + b + _NBUF, b)

    last0 = cpw - _NBUF
    for b in range(_NBUF):
        gather_wait(last0 + b, b)
        store_start(last0 + b, b)
    for b in range(_NBUF):
        store_wait(last0 + b, b)


def kernel(inputs, num_embeddings, features, embedding):
    batch, hist = inputs.shape
    feat = embedding.shape[1]
    total = batch * hist
    idx2d = inputs.reshape(total // _CHUNK, _CHUNK)

    cpw = (total // _CHUNK) // _NW
    mesh = plsc.VectorSubcoreMesh(core_axis_name="c", subcore_axis_name="s",
                                  num_cores=_NC, num_subcores=_NS)
    out = pl.kernel(
        functools.partial(_embed_gather, total, feat),
        out_type=jax.ShapeDtypeStruct((total, feat), jnp.float32),
        mesh=mesh,
        scratch_types=[
            pltpu.VMEM((cpw, _CHUNK), jnp.int32),
            pltpu.VMEM((_NBUF, _CHUNK, feat), jnp.float32),
            pltpu.SemaphoreType.DMA((_NBUF,)),
            pltpu.SemaphoreType.DMA((_NBUF,)),
        ],
        compiler_params=pltpu.CompilerParams(use_tc_tiling_on_sc=False),
    )(embedding, idx2d)
    return out.reshape(batch, hist, feat)


# speedup vs baseline: 1.3674x; 1.3648x over previous
import functools

import jax
import jax.numpy as jnp
from jax import lax
from jax.experimental import pallas as pl
from jax.experimental.pallas import tpu as pltpu
from jax.experimental.pallas import tpu_sc as plsc

_NC = 2
_NS = 16
_NW = _NC * _NS

_CHUNK = 128
_NBUF = 4
_PADF = 128


def _embed_gather(total_rows, table_hbm, idx_hbm, out_hbm,
                  idx_v, rows_v, gsems, ssems):
    chunks_total = total_rows // _CHUNK
    cpw = chunks_total // _NW
    wid = lax.axis_index("s") * _NC + lax.axis_index("c")
    chunk0 = wid * cpw

    pltpu.sync_copy(idx_hbm.at[pl.ds(chunk0, cpw)], idx_v)

    def gather_start(j, slot):
        pltpu.async_copy(table_hbm.at[idx_v.at[j]], rows_v.at[slot],
                         gsems.at[slot])

    def gather_wait(j, slot):
        pltpu.make_async_copy(table_hbm.at[idx_v.at[j]], rows_v.at[slot],
                              gsems.at[slot]).wait()

    def store_start(j, slot):
        base = (chunk0 + j) * _CHUNK
        pltpu.async_copy(rows_v.at[slot], out_hbm.at[pl.ds(base, _CHUNK)],
                         ssems.at[slot])

    def store_wait(j, slot):
        base = (chunk0 + j) * _CHUNK
        pltpu.make_async_copy(rows_v.at[slot],
                              out_hbm.at[pl.ds(base, _CHUNK)],
                              ssems.at[slot]).wait()

    for b in range(_NBUF):
        gather_start(b, b)

    @pl.loop(0, cpw - _NBUF, step=_NBUF)
    def _round(j0):
        for b in range(_NBUF):
            gather_wait(j0 + b, b)
            store_start(j0 + b, b)
        for b in range(_NBUF):
            store_wait(j0 + b, b)
            gather_start(j0 + b + _NBUF, b)

    last0 = cpw - _NBUF
    for b in range(_NBUF):
        gather_wait(last0 + b, b)
        store_start(last0 + b, b)
    for b in range(_NBUF):
        store_wait(last0 + b, b)


def kernel(inputs, num_embeddings, features, embedding):
    batch, hist = inputs.shape
    nrows, feat = embedding.shape
    total = batch * hist
    idx2d = inputs.reshape(total // _CHUNK, _CHUNK)
    pad_eye = jnp.eye(feat, _PADF, dtype=jnp.float32)
    table_p = lax.dot_general(embedding, pad_eye, (((1,), (0,)), ((), ())),
                              precision=lax.Precision.HIGHEST)

    cpw = (total // _CHUNK) // _NW
    mesh = plsc.VectorSubcoreMesh(core_axis_name="c", subcore_axis_name="s",
                                  num_cores=_NC, num_subcores=_NS)
    out = pl.kernel(
        functools.partial(_embed_gather, total),
        out_type=jax.ShapeDtypeStruct((total, _PADF), jnp.float32),
        mesh=mesh,
        scratch_types=[
            pltpu.VMEM((cpw, _CHUNK), jnp.int32),
            pltpu.VMEM((_NBUF, _CHUNK, _PADF), jnp.float32),
            pltpu.SemaphoreType.DMA((_NBUF,)),
            pltpu.SemaphoreType.DMA((_NBUF,)),
        ],
        compiler_params=pltpu.CompilerParams(use_tc_tiling_on_sc=False),
    )(table_p, idx2d)
    return out[:, :feat].reshape(batch, hist, feat)
